# baseline (speedup 1.0000x reference)
import functools

import jax
import jax.numpy as jnp
from jax import lax
from jax.experimental import pallas as pl
from jax.experimental.pallas import tpu as pltpu
from jax.experimental.pallas import tpu_sc as plsc

_EPS = 1e-12
_TC_GRID = 2

_D = 256
_LANES = 16
_NW = 32
_CHUNK = 64


def _tc_norm_kernel(z_ref, oz_ref):
    z = z_ref[...]
    n = jnp.sqrt(jnp.sum(z * z, axis=1, keepdims=True))
    oz_ref[...] = z * (1.0 / jnp.maximum(n, _EPS))


def _tc_normalize(z):
    bz = z.shape[0] // _TC_GRID
    return pl.pallas_call(
        _tc_norm_kernel,
        grid=(_TC_GRID,),
        in_specs=[pl.BlockSpec((bz, z.shape[1]), lambda i: (i, 0))],
        out_specs=pl.BlockSpec((bz, z.shape[1]), lambda i: (i, 0)),
        out_shape=jax.ShapeDtypeStruct(z.shape, z.dtype),
    )(z)


_GDN = lax.GatherDimensionNumbers(
    offset_dims=(), collapsed_slice_dims=(0,), start_index_map=(0,)
)


def _lane_perm(x, idx):
    return lax.gather(
        x, idx[:, None], dimension_numbers=_GDN, slice_sizes=(1,),
        mode=lax.GatherScatterMode.PROMISE_IN_BOUNDS,
    )


def _row_normalize_in_place(buf, r, perm_idx):
    accs = [jnp.zeros((_LANES,), jnp.float32) for _ in range(4)]
    chunks = []
    for j in range(_D // _LANES):
        x = buf[r, pl.ds(j * _LANES, _LANES)]
        chunks.append(x)
        accs[j % 4] = accs[j % 4] + x * x
    acc = (accs[0] + accs[1]) + (accs[2] + accs[3])
    for idx in perm_idx:
        acc = acc + _lane_perm(acc, idx)
    sv = acc
    i = plsc.bitcast(sv, jnp.int32)
    i = 0x5F3759DF - lax.shift_right_logical(i, 1)
    rs = plsc.bitcast(i, jnp.float32)
    half = sv * 0.5
    for _ in range(3):
        rs = rs * (1.5 - half * rs * rs)
    rs = jnp.minimum(rs, 1.0 / _EPS)
    for j in range(_D // _LANES):
        buf[r, pl.ds(j * _LANES, _LANES)] = chunks[j] * rs


def _sc_normalize(centroids):
    n_rows = centroids.shape[0]
    rows_per_w = n_rows // _NW
    n_chunks = rows_per_w // _CHUNK
    mesh = plsc.VectorSubcoreMesh(core_axis_name="c", subcore_axis_name="s")

    @functools.partial(
        pl.kernel,
        mesh=mesh,
        out_type=jax.ShapeDtypeStruct(centroids.shape, centroids.dtype),
        scratch_types=[
            pltpu.VMEM((_CHUNK, _D), jnp.float32),
            pltpu.VMEM((_CHUNK, _D), jnp.float32),
            pltpu.SemaphoreType.DMA,
            pltpu.SemaphoreType.DMA,
            pltpu.SemaphoreType.DMA,
            pltpu.SemaphoreType.DMA,
        ],
        compiler_params=pltpu.CompilerParams(needs_layout_passes=False),
    )
    def sc_norm(c_hbm, out_hbm, buf0, buf1, si0, si1, so0, so1):
        wid = lax.axis_index("s") * 2 + lax.axis_index("c")
        base = wid * rows_per_w
        iota = lax.iota(jnp.int32, _LANES)
        perm_idx = [lax.bitwise_xor(iota, k) for k in (1, 2, 4, 8)]
        bufs = (buf0, buf1)
        in_sems = (si0, si1)
        out_sems = (so0, so1)

        def in_copy(chunk):
            row0 = base + chunk * _CHUNK
            return pltpu.async_copy(
                c_hbm.at[pl.ds(row0, _CHUNK)], bufs[chunk % 2], in_sems[chunk % 2]
            )

        def out_copy(chunk):
            row0 = base + chunk * _CHUNK
            return pltpu.async_copy(
                bufs[chunk % 2], out_hbm.at[pl.ds(row0, _CHUNK)], out_sems[chunk % 2]
            )

        in_flight = {0: in_copy(0)}
        out_flight = {}
        for chunk in range(n_chunks):
            in_flight.pop(chunk).wait()
            if chunk + 1 < n_chunks:
                if chunk - 1 in out_flight:
                    out_flight.pop(chunk - 1).wait()
                in_flight[chunk + 1] = in_copy(chunk + 1)
            buf = bufs[chunk % 2]

            def body(r2, carry):
                _row_normalize_in_place(buf, r2 * 2, perm_idx)
                _row_normalize_in_place(buf, r2 * 2 + 1, perm_idx)
                return carry

            lax.fori_loop(0, _CHUNK // 2, body, 0)
            out_flight[chunk] = out_copy(chunk)
        for c in sorted(out_flight):
            out_flight.pop(c).wait()

    return sc_norm(centroids)


def kernel(z, centroids):
    return (z, _sc_normalize(centroids))


# speedup vs baseline: 2.4347x; 2.4347x over previous
import jax
import jax.numpy as jnp
from jax.experimental import pallas as pl

_EPS = 1e-12
_GRID = 2


def _norm_kernel(za_ref, zb_ref, ca_ref, cb_ref, oz_ref, oc_ref):
    za = za_ref[...]
    zb = zb_ref[...]
    n = jnp.sqrt(
        jnp.sum(za * za, axis=1, keepdims=True)
        + jnp.sum(zb * zb, axis=1, keepdims=True)
    )
    inv = 1.0 / jnp.maximum(n, _EPS)
    oz_ref[:, : za.shape[1]] = za * inv
    oz_ref[:, za.shape[1] :] = zb * inv
    ca = ca_ref[...]
    cb = cb_ref[...]
    m = jnp.sqrt(
        jnp.sum(ca * ca, axis=1, keepdims=True)
        + jnp.sum(cb * cb, axis=1, keepdims=True)
    )
    cinv = 1.0 / jnp.maximum(m, _EPS)
    oc_ref[:, : ca.shape[1]] = ca * cinv
    oc_ref[:, ca.shape[1] :] = cb * cinv


def kernel(z, centroids):
    bz = z.shape[0] // _GRID
    bc = centroids.shape[0] // _GRID
    d = z.shape[1]
    h = d // 2
    return pl.pallas_call(
        _norm_kernel,
        grid=(_GRID,),
        in_specs=[
            pl.BlockSpec((bz, h), lambda i: (i, 0)),
            pl.BlockSpec((bz, h), lambda i: (i, 1)),
            pl.BlockSpec((bc, h), lambda i: (i, 0)),
            pl.BlockSpec((bc, h), lambda i: (i, 1)),
        ],
        out_specs=[
            pl.BlockSpec((bz, d), lambda i: (i, 0)),
            pl.BlockSpec((bc, d), lambda i: (i, 0)),
        ],
        out_shape=[
            jax.ShapeDtypeStruct(z.shape, z.dtype),
            jax.ShapeDtypeStruct(centroids.shape, centroids.dtype),
        ],
    )(z, z, centroids, centroids)


# speedup vs baseline: 2.5466x; 1.0460x over previous
import jax
import jax.numpy as jnp
from jax.experimental import pallas as pl

_EPS = 1e-12
_GRID = 2


def _norm1(x):
    n = jnp.sqrt(jnp.sum(x * x, axis=1, keepdims=True))
    return x * (1.0 / jnp.maximum(n, _EPS))


def _norm_kernel(za_ref, zb_ref, ca_ref, cb_ref, oz_ref, oc_ref):
    oz_ref[0] = _norm1(za_ref[...])
    oz_ref[1] = _norm1(zb_ref[...])
    oc_ref[0] = _norm1(ca_ref[...])
    oc_ref[1] = _norm1(cb_ref[...])


def kernel(z, centroids):
    d = z.shape[1]
    hz = z.shape[0] // 2
    hc = centroids.shape[0] // 2
    bz = hz // _GRID
    bc = hc // _GRID
    oz3, oc3 = pl.pallas_call(
        _norm_kernel,
        grid=(_GRID,),
        in_specs=[
            pl.BlockSpec((bz, d), lambda i: (i, 0)),
            pl.BlockSpec((bz, d), lambda i: (i + _GRID, 0)),
            pl.BlockSpec((bc, d), lambda i: (i, 0)),
            pl.BlockSpec((bc, d), lambda i: (i + _GRID, 0)),
        ],
        out_specs=[
            pl.BlockSpec((2, bz, d), lambda i: (0, i, 0)),
            pl.BlockSpec((2, bc, d), lambda i: (0, i, 0)),
        ],
        out_shape=[
            jax.ShapeDtypeStruct((2, hz, d), z.dtype),
            jax.ShapeDtypeStruct((2, hc, d), centroids.dtype),
        ],
    )(z, z, centroids, centroids)
    return (oz3.reshape(z.shape), oc3.reshape(centroids.shape))


# speedup vs baseline: 2.6413x; 1.0372x over previous
import jax
import jax.numpy as jnp
from jax.experimental import pallas as pl

_EPS = 1e-12
_GRID = 2


def _norm_kernel(z_ref, c_ref, oz_ref, oc_ref):
    z = z_ref[...]
    n = jnp.sqrt(jnp.sum(z * z, axis=1, keepdims=True))
    oz_ref[...] = z * (1.0 / jnp.maximum(n, _EPS))
    c = c_ref[...]
    m = jnp.sqrt(jnp.sum(c * c, axis=1, keepdims=True))
    oc_ref[...] = c * (1.0 / jnp.maximum(m, _EPS))


def kernel(z, centroids):
    bz = z.shape[0] // _GRID
    bc = centroids.shape[0] // _GRID
    d = z.shape[1]
    return pl.pallas_call(
        _norm_kernel,
        grid=(_GRID,),
        in_specs=[
            pl.BlockSpec((bz, d), lambda i: (i, 0)),
            pl.BlockSpec((bc, d), lambda i: (i, 0)),
        ],
        out_specs=[
            pl.BlockSpec((bz, d), lambda i: (i, 0)),
            pl.BlockSpec((bc, d), lambda i: (i, 0)),
        ],
        out_shape=[
            jax.ShapeDtypeStruct(z.shape, z.dtype),
            jax.ShapeDtypeStruct(centroids.shape, centroids.dtype),
        ],
    )(z, centroids)
